# baseline (speedup 1.0000x reference)
import functools

import jax
import jax.numpy as jnp
from jax import lax
from jax.experimental import pallas as pl
from jax.experimental.pallas import tpu as pltpu
from jax.experimental.pallas import tpu_sc as plsc

N = 10000
E = 320000
H = 128
HH = H // 2
C = 10
G = 64

NC = 2
NS = 16
NW = NC * NS

K_A = 125
J_A = (E // NS) // K_A
NBUF = 5
GDEPTH = 3

K_D = 125
J_D = (E // NW) // K_D
DLANE = 16

RPT = 640
RPT_LAST = N - (NS - 1) * RPT

_BLK = 2000

_mesh = plsc.VectorSubcoreMesh(core_axis_name="c", subcore_axis_name="s")
_sc_params = pltpu.CompilerParams(use_tc_tiling_on_sc=False)


def _slice_copy(s, src_ref, dst_ref):
    @pl.when(s < NS - 1)
    def _():
        row0 = s * RPT
        pltpu.sync_copy(src_ref.at[pl.ds(row0, RPT)],
                        dst_ref.at[pl.ds(row0, RPT)])

    @pl.when(s == NS - 1)
    def _():
        row0 = (NS - 1) * RPT
        pltpu.sync_copy(src_ref.at[pl.ds(row0, RPT_LAST)],
                        dst_ref.at[pl.ds(row0, RPT_LAST)])


@functools.partial(
    pl.kernel,
    out_type=(jax.ShapeDtypeStruct((N, DLANE), jnp.float32),
              jax.ShapeDtypeStruct((N, DLANE), jnp.float32)),
    mesh=_mesh,
    scratch_types=[
        pltpu.VMEM((J_D, K_D), jnp.int32),
        pltpu.VMEM((K_D, DLANE), jnp.float32),
        pltpu.VMEM_SHARED((N, DLANE), jnp.float32),
        pltpu.SemaphoreType.DMA,
    ],
    compiler_params=_sc_params,
)
def _sc_degree(dst_hbm, zeros_hbm, out0_hbm, out1_hbm,
               dst_v, ones_v, acc_sh, sem):
    c = lax.axis_index("c")
    s = lax.axis_index("s")
    wid = c * NS + s
    pltpu.sync_copy(dst_hbm.at[wid], dst_v)

    @pl.loop(0, K_D)
    def _(k):
        ones_v[k, :] = jnp.full((DLANE,), 1.0, jnp.float32)

    _slice_copy(s, zeros_hbm, acc_sh)
    plsc.subcore_barrier()

    @pl.loop(0, J_D, step=8)
    def _(j0):
        copies = [
            pltpu.async_copy(ones_v, acc_sh.at[dst_v.at[j0 + u]], sem, add=True)
            for u in range(8)
        ]
        for cp in copies:
            cp.wait()

    plsc.subcore_barrier()

    @pl.when(c == 0)
    def _():
        _slice_copy(s, acc_sh, out0_hbm)

    @pl.when(c == 1)
    def _():
        _slice_copy(s, acc_sh, out1_hbm)


def _agg_loop(h_hbm, src_v, dst_v, rows_v, acc_sh, gsems, ssems):
    for u in range(GDEPTH):
        pltpu.async_copy(h_hbm.at[src_v.at[u]], rows_v.at[u], gsems[u])

    @pl.loop(0, J_A, step=NBUF)
    def _(j0):
        for u in range(NBUF):
            j = j0 + u
            nxt = j + GDEPTH
            bn = (u + GDEPTH) % NBUF

            @pl.when(nxt < J_A)
            def _():
                @pl.when(j >= NBUF - GDEPTH)
                def _():
                    pltpu.make_async_copy(
                        rows_v.at[bn], acc_sh.at[dst_v.at[j - (NBUF - GDEPTH)]],
                        ssems[bn]).wait()

                pltpu.async_copy(h_hbm.at[src_v.at[nxt]], rows_v.at[bn],
                                 gsems[bn])

            pltpu.make_async_copy(
                h_hbm.at[src_v.at[j]], rows_v.at[u], gsems[u]).wait()
            pltpu.async_copy(rows_v.at[u], acc_sh.at[dst_v.at[j]],
                             ssems[u], add=True)

    for u in range(NBUF):
        pltpu.make_async_copy(
            rows_v.at[u], acc_sh.at[dst_v.at[J_A - NBUF + u]],
            ssems[u]).wait()


@functools.partial(
    pl.kernel,
    out_type=(jax.ShapeDtypeStruct((N, HH), jnp.float32),
              jax.ShapeDtypeStruct((N, HH), jnp.float32)),
    mesh=_mesh,
    scratch_types=[
        pltpu.VMEM((J_A, K_A), jnp.int32),
        pltpu.VMEM((J_A, K_A), jnp.int32),
        pltpu.VMEM((NBUF, K_A, HH), jnp.float32),
        pltpu.VMEM_SHARED((N, HH), jnp.float32),
    ] + [pltpu.SemaphoreType.DMA] * (2 * NBUF),
    compiler_params=_sc_params,
)
def _sc_aggregate(h0_hbm, h1_hbm, src_hbm, dst_hbm, zeros_hbm,
                  out0_hbm, out1_hbm, src_v, dst_v, rows_v, acc_sh, *sems):
    gsems, ssems = sems[:NBUF], sems[NBUF:]
    c = lax.axis_index("c")
    s = lax.axis_index("s")
    pltpu.sync_copy(src_hbm.at[s], src_v)
    pltpu.sync_copy(dst_hbm.at[s], dst_v)
    _slice_copy(s, zeros_hbm, acc_sh)
    plsc.subcore_barrier()

    @pl.when(c == 0)
    def _():
        _agg_loop(h0_hbm, src_v, dst_v, rows_v, acc_sh, gsems, ssems)

    @pl.when(c == 1)
    def _():
        _agg_loop(h1_hbm, src_v, dst_v, rows_v, acc_sh, gsems, ssems)

    plsc.subcore_barrier()

    @pl.when(c == 0)
    def _():
        _slice_copy(s, acc_sh, out0_hbm)

    @pl.when(c == 1)
    def _():
        _slice_copy(s, acc_sh, out1_hbm)


def _dinv(d0_ref, d1_ref):
    return lax.rsqrt(d0_ref[:, 0:1] + d1_ref[:, 0:1] + 1.0)


def _mm_body(x_ref, w_ref, o_ref):
    o_ref[...] = jnp.dot(x_ref[...], w_ref[...],
                         preferred_element_type=jnp.float32)


def _tc_matmul(x, w):
    return pl.pallas_call(
        _mm_body,
        grid=(N // _BLK,),
        in_specs=[pl.BlockSpec((_BLK, H), lambda i: (i, 0)),
                  pl.BlockSpec((H, H), lambda i: (0, 0))],
        out_specs=pl.BlockSpec((_BLK, H), lambda i: (i, 0)),
        out_shape=jax.ShapeDtypeStruct((N, H), jnp.float32),
    )(x, w)


_half_spec = pl.BlockSpec((_BLK, HH), lambda i: (i, 0))
_deg_spec = pl.BlockSpec((_BLK, DLANE), lambda i: (i, 0))
_half_out = (jax.ShapeDtypeStruct((N, HH), jnp.float32),
             jax.ShapeDtypeStruct((N, HH), jnp.float32))


def _scale_body(h_ref, d0_ref, d1_ref, o0_ref, o1_ref):
    hp = h_ref[...] * _dinv(d0_ref, d1_ref)
    o0_ref[...] = hp[:, :HH]
    o1_ref[...] = hp[:, HH:]


def _tc_scale(h, d0, d1):
    return pl.pallas_call(
        _scale_body,
        grid=(N // _BLK,),
        in_specs=[pl.BlockSpec((_BLK, H), lambda i: (i, 0)),
                  _deg_spec, _deg_spec],
        out_specs=[_half_spec, _half_spec],
        out_shape=_half_out,
    )(h, d0, d1)


def _mid_body(p0_ref, p1_ref, hp0_ref, hp1_ref, d0_ref, d1_ref, b_ref, w_ref,
              o0_ref, o1_ref):
    dinv = _dinv(d0_ref, d1_ref)
    t = jnp.concatenate([p0_ref[...] + hp0_ref[...],
                         p1_ref[...] + hp1_ref[...]], axis=1)
    t = jnp.maximum(t * dinv + b_ref[...], 0.0)
    h2 = jnp.dot(t, w_ref[...], preferred_element_type=jnp.float32) * dinv
    o0_ref[...] = h2[:, :HH]
    o1_ref[...] = h2[:, HH:]


def _tc_mid(p0, p1, hp0, hp1, d0, d1, b, w):
    return pl.pallas_call(
        _mid_body,
        grid=(N // _BLK,),
        in_specs=[_half_spec, _half_spec, _half_spec, _half_spec,
                  _deg_spec, _deg_spec,
                  pl.BlockSpec((1, H), lambda i: (0, 0)),
                  pl.BlockSpec((H, H), lambda i: (0, 0))],
        out_specs=[_half_spec, _half_spec],
        out_shape=_half_out,
    )(p0, p1, hp0, hp1, d0, d1, b, w)


def _final_body(p0_ref, p1_ref, hp0_ref, hp1_ref, d0_ref, d1_ref, b_ref,
                batch_ref, wc_ref, bc_ref, o_ref, sums_ref, cnt_ref):
    i = pl.program_id(0)

    @pl.when(i == 0)
    def _():
        sums_ref[...] = jnp.zeros_like(sums_ref)
        cnt_ref[...] = jnp.zeros_like(cnt_ref)

    dinv = _dinv(d0_ref, d1_ref)
    t = jnp.concatenate([p0_ref[...] + hp0_ref[...],
                         p1_ref[...] + hp1_ref[...]], axis=1)
    x4 = jnp.maximum(t * dinv + b_ref[...], 0.0)
    bvec = batch_ref[0, 0, :]
    gids = lax.broadcasted_iota(jnp.int32, (G, _BLK), 0)
    sel = (gids == bvec[None, :]).astype(jnp.float32)
    sums_ref[...] += jnp.dot(sel, x4, preferred_element_type=jnp.float32)
    cnt_ref[...] += jnp.broadcast_to(
        jnp.sum(sel, axis=1, keepdims=True), cnt_ref.shape)

    @pl.when(i == pl.num_programs(0) - 1)
    def _():
        pooled = sums_ref[...] / jnp.maximum(cnt_ref[...], 1.0)
        o_ref[...] = jnp.dot(pooled, wc_ref[...],
                             preferred_element_type=jnp.float32) + bc_ref[...]


def _tc_final(p0, p1, hp0, hp1, d0, d1, b, batch3, wc, bcr):
    return pl.pallas_call(
        _final_body,
        grid=(N // _BLK,),
        in_specs=[_half_spec, _half_spec, _half_spec, _half_spec,
                  _deg_spec, _deg_spec,
                  pl.BlockSpec((1, H), lambda i: (0, 0)),
                  pl.BlockSpec((1, 1, _BLK), lambda i: (i, 0, 0)),
                  pl.BlockSpec((H, C), lambda i: (0, 0)),
                  pl.BlockSpec((1, C), lambda i: (0, 0))],
        out_specs=pl.BlockSpec((G, C), lambda i: (0, 0)),
        out_shape=jax.ShapeDtypeStruct((G, C), jnp.float32),
        scratch_shapes=[pltpu.VMEM((G, H), jnp.float32),
                        pltpu.VMEM((G, H), jnp.float32)],
    )(p0, p1, hp0, hp1, d0, d1, b, batch3, wc, bcr)


def kernel(x, edge_index, batch, W1, b1, W2, b2, W3, b3, Wc, bc):
    srcw = edge_index[0].reshape(NS, J_A, K_A)
    dstw = edge_index[1].reshape(NS, J_A, K_A)
    dstd = edge_index[1].reshape(NW, J_D, K_D)

    zeros_h = jnp.zeros((N, HH), jnp.float32)
    zeros_d = jnp.zeros((N, DLANE), jnp.float32)
    batch3 = batch.reshape(N // _BLK, 1, _BLK)
    b1r, b2r, b3r = b1.reshape(1, H), b2.reshape(1, H), b3.reshape(1, H)
    bcr = bc.reshape(1, C)

    d0, d1 = _sc_degree(dstd, zeros_d)
    h1 = _tc_matmul(x, W1)
    h10, h11 = _tc_scale(h1, d0, d1)
    p10, p11 = _sc_aggregate(h10, h11, srcw, dstw, zeros_h)
    h20, h21 = _tc_mid(p10, p11, h10, h11, d0, d1, b1r, W2)
    p20, p21 = _sc_aggregate(h20, h21, srcw, dstw, zeros_h)
    h30, h31 = _tc_mid(p20, p21, h20, h21, d0, d1, b2r, W3)
    p30, p31 = _sc_aggregate(h30, h31, srcw, dstw, zeros_h)
    return _tc_final(p30, p31, h30, h31, d0, d1, b3r, batch3, Wc, bcr)


# speedup vs baseline: 1.0103x; 1.0103x over previous
import functools

import jax
import jax.numpy as jnp
from jax import lax
from jax.experimental import pallas as pl
from jax.experimental.pallas import tpu as pltpu
from jax.experimental.pallas import tpu_sc as plsc

N = 10000
E = 320000
H = 128
HH = H // 2
C = 10
G = 64

NC = 2
NS = 16
NW = NC * NS

K_A = 250
J_A = (E // NS) // K_A
NBUF = 3
GDEPTH = 2

K_D = 250
J_D = (E // NW) // K_D
DLANE = 16

RPT = 640
RPT_LAST = N - (NS - 1) * RPT

_BLK = 2000

_mesh = plsc.VectorSubcoreMesh(core_axis_name="c", subcore_axis_name="s")
_sc_params = pltpu.CompilerParams(use_tc_tiling_on_sc=False)


def _slice_copy(s, src_ref, dst_ref):
    @pl.when(s < NS - 1)
    def _():
        row0 = s * RPT
        pltpu.sync_copy(src_ref.at[pl.ds(row0, RPT)],
                        dst_ref.at[pl.ds(row0, RPT)])

    @pl.when(s == NS - 1)
    def _():
        row0 = (NS - 1) * RPT
        pltpu.sync_copy(src_ref.at[pl.ds(row0, RPT_LAST)],
                        dst_ref.at[pl.ds(row0, RPT_LAST)])


@functools.partial(
    pl.kernel,
    out_type=(jax.ShapeDtypeStruct((N, DLANE), jnp.float32),
              jax.ShapeDtypeStruct((N, DLANE), jnp.float32)),
    mesh=_mesh,
    scratch_types=[
        pltpu.VMEM((J_D, K_D), jnp.int32),
        pltpu.VMEM((K_D, DLANE), jnp.float32),
        pltpu.VMEM_SHARED((N, DLANE), jnp.float32),
        pltpu.SemaphoreType.DMA,
    ],
    compiler_params=_sc_params,
)
def _sc_degree(dst_hbm, zeros_hbm, out0_hbm, out1_hbm,
               dst_v, ones_v, acc_sh, sem):
    c = lax.axis_index("c")
    s = lax.axis_index("s")
    wid = c * NS + s
    pltpu.sync_copy(dst_hbm.at[wid], dst_v)

    @pl.loop(0, K_D)
    def _(k):
        ones_v[k, :] = jnp.full((DLANE,), 1.0, jnp.float32)

    _slice_copy(s, zeros_hbm, acc_sh)
    plsc.subcore_barrier()

    @pl.loop(0, J_D, step=8)
    def _(j0):
        copies = [
            pltpu.async_copy(ones_v, acc_sh.at[dst_v.at[j0 + u]], sem, add=True)
            for u in range(8)
        ]
        for cp in copies:
            cp.wait()

    plsc.subcore_barrier()

    @pl.when(c == 0)
    def _():
        _slice_copy(s, acc_sh, out0_hbm)

    @pl.when(c == 1)
    def _():
        _slice_copy(s, acc_sh, out1_hbm)


def _agg_loop(h_hbm, src_v, dst_v, rows_v, acc_sh, gsems, ssems):
    for u in range(GDEPTH):
        pltpu.async_copy(h_hbm.at[src_v.at[u]], rows_v.at[u], gsems[u])

    @pl.loop(0, J_A - GDEPTH, step=NBUF)
    def _(j0):
        for u in range(NBUF):
            j = j0 + u
            bn = (u + GDEPTH) % NBUF

            @pl.when(j >= NBUF - GDEPTH)
            def _():
                pltpu.make_async_copy(
                    rows_v.at[bn], acc_sh.at[dst_v.at[j - (NBUF - GDEPTH)]],
                    ssems[bn]).wait()

            pltpu.async_copy(h_hbm.at[src_v.at[j + GDEPTH]], rows_v.at[bn],
                             gsems[bn])
            pltpu.make_async_copy(
                h_hbm.at[src_v.at[j]], rows_v.at[u], gsems[u]).wait()
            pltpu.async_copy(rows_v.at[u], acc_sh.at[dst_v.at[j]],
                             ssems[u], add=True)

    for t in range(J_A - GDEPTH, J_A):
        u = t % NBUF
        pltpu.make_async_copy(
            h_hbm.at[src_v.at[t]], rows_v.at[u], gsems[u]).wait()
        pltpu.async_copy(rows_v.at[u], acc_sh.at[dst_v.at[t]],
                         ssems[u], add=True)

    for t in range(J_A - NBUF, J_A):
        u = t % NBUF
        pltpu.make_async_copy(
            rows_v.at[u], acc_sh.at[dst_v.at[t]], ssems[u]).wait()


@functools.partial(
    pl.kernel,
    out_type=(jax.ShapeDtypeStruct((N, HH), jnp.float32),
              jax.ShapeDtypeStruct((N, HH), jnp.float32)),
    mesh=_mesh,
    scratch_types=[
        pltpu.VMEM((J_A, K_A), jnp.int32),
        pltpu.VMEM((J_A, K_A), jnp.int32),
        pltpu.VMEM((NBUF, K_A, HH), jnp.float32),
        pltpu.VMEM_SHARED((N, HH), jnp.float32),
    ] + [pltpu.SemaphoreType.DMA] * (2 * NBUF),
    compiler_params=_sc_params,
)
def _sc_aggregate(h0_hbm, h1_hbm, src_hbm, dst_hbm, zeros_hbm,
                  out0_hbm, out1_hbm, src_v, dst_v, rows_v, acc_sh, *sems):
    gsems, ssems = sems[:NBUF], sems[NBUF:]
    c = lax.axis_index("c")
    s = lax.axis_index("s")
    pltpu.sync_copy(src_hbm.at[s], src_v)
    pltpu.sync_copy(dst_hbm.at[s], dst_v)
    _slice_copy(s, zeros_hbm, acc_sh)
    plsc.subcore_barrier()

    @pl.when(c == 0)
    def _():
        _agg_loop(h0_hbm, src_v, dst_v, rows_v, acc_sh, gsems, ssems)

    @pl.when(c == 1)
    def _():
        _agg_loop(h1_hbm, src_v, dst_v, rows_v, acc_sh, gsems, ssems)

    plsc.subcore_barrier()

    @pl.when(c == 0)
    def _():
        _slice_copy(s, acc_sh, out0_hbm)

    @pl.when(c == 1)
    def _():
        _slice_copy(s, acc_sh, out1_hbm)


def _dinv(d0_ref, d1_ref):
    return lax.rsqrt(d0_ref[:, 0:1] + d1_ref[:, 0:1] + 1.0)


def _mm_body(x_ref, w_ref, o_ref):
    o_ref[...] = jnp.dot(x_ref[...], w_ref[...],
                         preferred_element_type=jnp.float32)


def _tc_matmul(x, w):
    return pl.pallas_call(
        _mm_body,
        grid=(N // _BLK,),
        in_specs=[pl.BlockSpec((_BLK, H), lambda i: (i, 0)),
                  pl.BlockSpec((H, H), lambda i: (0, 0))],
        out_specs=pl.BlockSpec((_BLK, H), lambda i: (i, 0)),
        out_shape=jax.ShapeDtypeStruct((N, H), jnp.float32),
    )(x, w)


_half_spec = pl.BlockSpec((_BLK, HH), lambda i: (i, 0))
_deg_spec = pl.BlockSpec((_BLK, DLANE), lambda i: (i, 0))
_half_out = (jax.ShapeDtypeStruct((N, HH), jnp.float32),
             jax.ShapeDtypeStruct((N, HH), jnp.float32))


def _scale_body(h_ref, d0_ref, d1_ref, o0_ref, o1_ref):
    hp = h_ref[...] * _dinv(d0_ref, d1_ref)
    o0_ref[...] = hp[:, :HH]
    o1_ref[...] = hp[:, HH:]


def _tc_scale(h, d0, d1):
    return pl.pallas_call(
        _scale_body,
        grid=(N // _BLK,),
        in_specs=[pl.BlockSpec((_BLK, H), lambda i: (i, 0)),
                  _deg_spec, _deg_spec],
        out_specs=[_half_spec, _half_spec],
        out_shape=_half_out,
    )(h, d0, d1)


def _mid_body(p0_ref, p1_ref, hp0_ref, hp1_ref, d0_ref, d1_ref, b_ref, w_ref,
              o0_ref, o1_ref):
    dinv = _dinv(d0_ref, d1_ref)
    t = jnp.concatenate([p0_ref[...] + hp0_ref[...],
                         p1_ref[...] + hp1_ref[...]], axis=1)
    t = jnp.maximum(t * dinv + b_ref[...], 0.0)
    h2 = jnp.dot(t, w_ref[...], preferred_element_type=jnp.float32) * dinv
    o0_ref[...] = h2[:, :HH]
    o1_ref[...] = h2[:, HH:]


def _tc_mid(p0, p1, hp0, hp1, d0, d1, b, w):
    return pl.pallas_call(
        _mid_body,
        grid=(N // _BLK,),
        in_specs=[_half_spec, _half_spec, _half_spec, _half_spec,
                  _deg_spec, _deg_spec,
                  pl.BlockSpec((1, H), lambda i: (0, 0)),
                  pl.BlockSpec((H, H), lambda i: (0, 0))],
        out_specs=[_half_spec, _half_spec],
        out_shape=_half_out,
    )(p0, p1, hp0, hp1, d0, d1, b, w)


def _final_body(p0_ref, p1_ref, hp0_ref, hp1_ref, d0_ref, d1_ref, b_ref,
                batch_ref, wc_ref, bc_ref, o_ref, sums_ref, cnt_ref):
    i = pl.program_id(0)

    @pl.when(i == 0)
    def _():
        sums_ref[...] = jnp.zeros_like(sums_ref)
        cnt_ref[...] = jnp.zeros_like(cnt_ref)

    dinv = _dinv(d0_ref, d1_ref)
    t = jnp.concatenate([p0_ref[...] + hp0_ref[...],
                         p1_ref[...] + hp1_ref[...]], axis=1)
    x4 = jnp.maximum(t * dinv + b_ref[...], 0.0)
    bvec = batch_ref[0, 0, :]
    gids = lax.broadcasted_iota(jnp.int32, (G, _BLK), 0)
    sel = (gids == bvec[None, :]).astype(jnp.float32)
    sums_ref[...] += jnp.dot(sel, x4, preferred_element_type=jnp.float32)
    cnt_ref[...] += jnp.broadcast_to(
        jnp.sum(sel, axis=1, keepdims=True), cnt_ref.shape)

    @pl.when(i == pl.num_programs(0) - 1)
    def _():
        pooled = sums_ref[...] / jnp.maximum(cnt_ref[...], 1.0)
        o_ref[...] = jnp.dot(pooled, wc_ref[...],
                             preferred_element_type=jnp.float32) + bc_ref[...]


def _tc_final(p0, p1, hp0, hp1, d0, d1, b, batch3, wc, bcr):
    return pl.pallas_call(
        _final_body,
        grid=(N // _BLK,),
        in_specs=[_half_spec, _half_spec, _half_spec, _half_spec,
                  _deg_spec, _deg_spec,
                  pl.BlockSpec((1, H), lambda i: (0, 0)),
                  pl.BlockSpec((1, 1, _BLK), lambda i: (i, 0, 0)),
                  pl.BlockSpec((H, C), lambda i: (0, 0)),
                  pl.BlockSpec((1, C), lambda i: (0, 0))],
        out_specs=pl.BlockSpec((G, C), lambda i: (0, 0)),
        out_shape=jax.ShapeDtypeStruct((G, C), jnp.float32),
        scratch_shapes=[pltpu.VMEM((G, H), jnp.float32),
                        pltpu.VMEM((G, H), jnp.float32)],
    )(p0, p1, hp0, hp1, d0, d1, b, batch3, wc, bcr)


def kernel(x, edge_index, batch, W1, b1, W2, b2, W3, b3, Wc, bc):
    srcw = edge_index[0].reshape(NS, J_A, K_A)
    dstw = edge_index[1].reshape(NS, J_A, K_A)
    dstd = edge_index[1].reshape(NW, J_D, K_D)

    zeros_h = jnp.zeros((N, HH), jnp.float32)
    zeros_d = jnp.zeros((N, DLANE), jnp.float32)
    batch3 = batch.reshape(N // _BLK, 1, _BLK)
    b1r, b2r, b3r = b1.reshape(1, H), b2.reshape(1, H), b3.reshape(1, H)
    bcr = bc.reshape(1, C)

    d0, d1 = _sc_degree(dstd, zeros_d)
    h1 = _tc_matmul(x, W1)
    h10, h11 = _tc_scale(h1, d0, d1)
    p10, p11 = _sc_aggregate(h10, h11, srcw, dstw, zeros_h)
    h20, h21 = _tc_mid(p10, p11, h10, h11, d0, d1, b1r, W2)
    p20, p21 = _sc_aggregate(h20, h21, srcw, dstw, zeros_h)
    h30, h31 = _tc_mid(p20, p21, h20, h21, d0, d1, b2r, W3)
    p30, p31 = _sc_aggregate(h30, h31, srcw, dstw, zeros_h)
    return _tc_final(p30, p31, h30, h31, d0, d1, b3r, batch3, Wc, bcr)


# speedup vs baseline: 1.3297x; 1.3162x over previous
import functools

import jax
import jax.numpy as jnp
from jax import lax
from jax.experimental import pallas as pl
from jax.experimental.pallas import tpu as pltpu
from jax.experimental.pallas import tpu_sc as plsc

N = 10000
E = 320000
H = 128
HH = H // 2
C = 10
G = 64

NC = 2
NS = 16
NW = NC * NS

K_A = 250
J_A = (E // NS) // K_A
NBUF = 3
GDEPTH = 2

K_D = 250
J_D = (E // NW) // K_D
DLANE = 16

RPT = 640
RPT_LAST = N - (NS - 1) * RPT

_BLK = 2000

_mesh = plsc.VectorSubcoreMesh(core_axis_name="c", subcore_axis_name="s")
_sc_params = pltpu.CompilerParams(use_tc_tiling_on_sc=False)


def _slice_copy(s, src_ref, dst_ref):
    @pl.when(s < NS - 1)
    def _():
        row0 = s * RPT
        pltpu.sync_copy(src_ref.at[pl.ds(row0, RPT)],
                        dst_ref.at[pl.ds(row0, RPT)])

    @pl.when(s == NS - 1)
    def _():
        row0 = (NS - 1) * RPT
        pltpu.sync_copy(src_ref.at[pl.ds(row0, RPT_LAST)],
                        dst_ref.at[pl.ds(row0, RPT_LAST)])


@functools.partial(
    pl.kernel,
    out_type=(jax.ShapeDtypeStruct((N, DLANE), jnp.float32),
              jax.ShapeDtypeStruct((N, DLANE), jnp.float32)),
    mesh=_mesh,
    scratch_types=[
        pltpu.VMEM((J_D, K_D), jnp.int32),
        pltpu.VMEM((K_D, DLANE), jnp.float32),
        pltpu.VMEM_SHARED((N, DLANE), jnp.float32),
        pltpu.SemaphoreType.DMA,
    ],
    compiler_params=_sc_params,
)
def _sc_degree(dst_hbm, zeros_hbm, out0_hbm, out1_hbm,
               dst_v, ones_v, acc_sh, sem):
    c = lax.axis_index("c")
    s = lax.axis_index("s")
    wid = c * NS + s
    pltpu.sync_copy(dst_hbm.at[wid], dst_v)

    @pl.loop(0, K_D)
    def _(k):
        ones_v[k, :] = jnp.full((DLANE,), 1.0, jnp.float32)

    _slice_copy(s, zeros_hbm, acc_sh)
    plsc.subcore_barrier()

    @pl.loop(0, J_D, step=8)
    def _(j0):
        copies = [
            pltpu.async_copy(ones_v, acc_sh.at[dst_v.at[j0 + u]], sem, add=True)
            for u in range(8)
        ]
        for cp in copies:
            cp.wait()

    plsc.subcore_barrier()

    @pl.when(c == 0)
    def _():
        _slice_copy(s, acc_sh, out0_hbm)

    @pl.when(c == 1)
    def _():
        _slice_copy(s, acc_sh, out1_hbm)


def _agg_loop(h_hbm, src_v, dst_v, rows_v, acc_sh, gsems, ssems):
    for u in range(GDEPTH):
        pltpu.async_copy(h_hbm.at[src_v.at[u]], rows_v.at[u], gsems[u])

    @pl.loop(0, J_A - GDEPTH, step=NBUF)
    def _(j0):
        for u in range(NBUF):
            j = j0 + u
            bn = (u + GDEPTH) % NBUF

            @pl.when(j >= NBUF - GDEPTH)
            def _():
                pltpu.make_async_copy(
                    rows_v.at[bn], acc_sh.at[dst_v.at[j - (NBUF - GDEPTH)]],
                    ssems[bn]).wait()

            pltpu.async_copy(h_hbm.at[src_v.at[j + GDEPTH]], rows_v.at[bn],
                             gsems[bn])
            pltpu.make_async_copy(
                h_hbm.at[src_v.at[j]], rows_v.at[u], gsems[u]).wait()
            pltpu.async_copy(rows_v.at[u], acc_sh.at[dst_v.at[j]],
                             ssems[u], add=True)

    for t in range(J_A - GDEPTH, J_A):
        u = t % NBUF
        pltpu.make_async_copy(
            h_hbm.at[src_v.at[t]], rows_v.at[u], gsems[u]).wait()
        pltpu.async_copy(rows_v.at[u], acc_sh.at[dst_v.at[t]],
                         ssems[u], add=True)

    for t in range(J_A - NBUF, J_A):
        u = t % NBUF
        pltpu.make_async_copy(
            rows_v.at[u], acc_sh.at[dst_v.at[t]], ssems[u]).wait()


@functools.partial(
    pl.kernel,
    out_type=(jax.ShapeDtypeStruct((N, HH), jnp.bfloat16),
              jax.ShapeDtypeStruct((N, HH), jnp.bfloat16)),
    mesh=_mesh,
    scratch_types=[
        pltpu.VMEM((J_A, K_A), jnp.int32),
        pltpu.VMEM((J_A, K_A), jnp.int32),
        pltpu.VMEM((NBUF, K_A, HH), jnp.bfloat16),
        pltpu.VMEM_SHARED((N, HH), jnp.bfloat16),
    ] + [pltpu.SemaphoreType.DMA] * (2 * NBUF),
    compiler_params=_sc_params,
)
def _sc_aggregate(h0_hbm, h1_hbm, src_hbm, dst_hbm, zeros_hbm,
                  out0_hbm, out1_hbm, src_v, dst_v, rows_v, acc_sh, *sems):
    gsems, ssems = sems[:NBUF], sems[NBUF:]
    c = lax.axis_index("c")
    s = lax.axis_index("s")
    pltpu.sync_copy(src_hbm.at[s], src_v)
    pltpu.sync_copy(dst_hbm.at[s], dst_v)
    _slice_copy(s, zeros_hbm, acc_sh)
    plsc.subcore_barrier()

    @pl.when(c == 0)
    def _():
        _agg_loop(h0_hbm, src_v, dst_v, rows_v, acc_sh, gsems, ssems)

    @pl.when(c == 1)
    def _():
        _agg_loop(h1_hbm, src_v, dst_v, rows_v, acc_sh, gsems, ssems)

    plsc.subcore_barrier()

    @pl.when(c == 0)
    def _():
        _slice_copy(s, acc_sh, out0_hbm)

    @pl.when(c == 1)
    def _():
        _slice_copy(s, acc_sh, out1_hbm)


def _dinv(d0_ref, d1_ref):
    return lax.rsqrt(d0_ref[:, 0:1] + d1_ref[:, 0:1] + 1.0)


def _mm_body(x_ref, w_ref, o_ref):
    o_ref[...] = jnp.dot(x_ref[...], w_ref[...],
                         preferred_element_type=jnp.float32)


def _tc_matmul(x, w):
    return pl.pallas_call(
        _mm_body,
        grid=(N // _BLK,),
        in_specs=[pl.BlockSpec((_BLK, H), lambda i: (i, 0)),
                  pl.BlockSpec((H, H), lambda i: (0, 0))],
        out_specs=pl.BlockSpec((_BLK, H), lambda i: (i, 0)),
        out_shape=jax.ShapeDtypeStruct((N, H), jnp.float32),
    )(x, w)


_half_spec = pl.BlockSpec((_BLK, HH), lambda i: (i, 0))
_deg_spec = pl.BlockSpec((_BLK, DLANE), lambda i: (i, 0))
_half_out = (jax.ShapeDtypeStruct((N, HH), jnp.bfloat16),
             jax.ShapeDtypeStruct((N, HH), jnp.bfloat16))


def _scale_body(h_ref, d0_ref, d1_ref, o0_ref, o1_ref):
    hp = (h_ref[...] * _dinv(d0_ref, d1_ref)).astype(jnp.bfloat16)
    o0_ref[...] = hp[:, :HH]
    o1_ref[...] = hp[:, HH:]


def _tc_scale(h, d0, d1):
    return pl.pallas_call(
        _scale_body,
        grid=(N // _BLK,),
        in_specs=[pl.BlockSpec((_BLK, H), lambda i: (i, 0)),
                  _deg_spec, _deg_spec],
        out_specs=[_half_spec, _half_spec],
        out_shape=_half_out,
    )(h, d0, d1)


def _mid_body(p0_ref, p1_ref, hp0_ref, hp1_ref, d0_ref, d1_ref, b_ref, w_ref,
              o0_ref, o1_ref):
    dinv = _dinv(d0_ref, d1_ref)
    t = jnp.concatenate([p0_ref[...] + hp0_ref[...],
                         p1_ref[...] + hp1_ref[...]], axis=1
                        ).astype(jnp.float32)
    t = jnp.maximum(t * dinv + b_ref[...], 0.0)
    h2 = (jnp.dot(t, w_ref[...], preferred_element_type=jnp.float32)
          * dinv).astype(jnp.bfloat16)
    o0_ref[...] = h2[:, :HH]
    o1_ref[...] = h2[:, HH:]


def _tc_mid(p0, p1, hp0, hp1, d0, d1, b, w):
    return pl.pallas_call(
        _mid_body,
        grid=(N // _BLK,),
        in_specs=[_half_spec, _half_spec, _half_spec, _half_spec,
                  _deg_spec, _deg_spec,
                  pl.BlockSpec((1, H), lambda i: (0, 0)),
                  pl.BlockSpec((H, H), lambda i: (0, 0))],
        out_specs=[_half_spec, _half_spec],
        out_shape=_half_out,
    )(p0, p1, hp0, hp1, d0, d1, b, w)


def _final_body(p0_ref, p1_ref, hp0_ref, hp1_ref, d0_ref, d1_ref, b_ref,
                batch_ref, wc_ref, bc_ref, o_ref, sums_ref, cnt_ref):
    i = pl.program_id(0)

    @pl.when(i == 0)
    def _():
        sums_ref[...] = jnp.zeros_like(sums_ref)
        cnt_ref[...] = jnp.zeros_like(cnt_ref)

    dinv = _dinv(d0_ref, d1_ref)
    t = jnp.concatenate([p0_ref[...] + hp0_ref[...],
                         p1_ref[...] + hp1_ref[...]], axis=1
                        ).astype(jnp.float32)
    x4 = jnp.maximum(t * dinv + b_ref[...], 0.0)
    bvec = batch_ref[0, 0, :]
    gids = lax.broadcasted_iota(jnp.int32, (G, _BLK), 0)
    sel = (gids == bvec[None, :]).astype(jnp.float32)
    sums_ref[...] += jnp.dot(sel, x4, preferred_element_type=jnp.float32)
    cnt_ref[...] += jnp.broadcast_to(
        jnp.sum(sel, axis=1, keepdims=True), cnt_ref.shape)

    @pl.when(i == pl.num_programs(0) - 1)
    def _():
        pooled = sums_ref[...] / jnp.maximum(cnt_ref[...], 1.0)
        o_ref[...] = jnp.dot(pooled, wc_ref[...],
                             preferred_element_type=jnp.float32) + bc_ref[...]


def _tc_final(p0, p1, hp0, hp1, d0, d1, b, batch3, wc, bcr):
    return pl.pallas_call(
        _final_body,
        grid=(N // _BLK,),
        in_specs=[_half_spec, _half_spec, _half_spec, _half_spec,
                  _deg_spec, _deg_spec,
                  pl.BlockSpec((1, H), lambda i: (0, 0)),
                  pl.BlockSpec((1, 1, _BLK), lambda i: (i, 0, 0)),
                  pl.BlockSpec((H, C), lambda i: (0, 0)),
                  pl.BlockSpec((1, C), lambda i: (0, 0))],
        out_specs=pl.BlockSpec((G, C), lambda i: (0, 0)),
        out_shape=jax.ShapeDtypeStruct((G, C), jnp.float32),
        scratch_shapes=[pltpu.VMEM((G, H), jnp.float32),
                        pltpu.VMEM((G, H), jnp.float32)],
    )(p0, p1, hp0, hp1, d0, d1, b, batch3, wc, bcr)


def kernel(x, edge_index, batch, W1, b1, W2, b2, W3, b3, Wc, bc):
    srcw = edge_index[0].reshape(NS, J_A, K_A)
    dstw = edge_index[1].reshape(NS, J_A, K_A)
    dstd = edge_index[1].reshape(NW, J_D, K_D)

    zeros_h = jnp.zeros((N, HH), jnp.bfloat16)
    zeros_d = jnp.zeros((N, DLANE), jnp.float32)
    batch3 = batch.reshape(N // _BLK, 1, _BLK)
    b1r, b2r, b3r = b1.reshape(1, H), b2.reshape(1, H), b3.reshape(1, H)
    bcr = bc.reshape(1, C)

    d0, d1 = _sc_degree(dstd, zeros_d)
    h1 = _tc_matmul(x, W1)
    h10, h11 = _tc_scale(h1, d0, d1)
    p10, p11 = _sc_aggregate(h10, h11, srcw, dstw, zeros_h)
    h20, h21 = _tc_mid(p10, p11, h10, h11, d0, d1, b1r, W2)
    p20, p21 = _sc_aggregate(h20, h21, srcw, dstw, zeros_h)
    h30, h31 = _tc_mid(p20, p21, h20, h21, d0, d1, b2r, W3)
    p30, p31 = _sc_aggregate(h30, h31, srcw, dstw, zeros_h)
    return _tc_final(p30, p31, h30, h31, d0, d1, b3r, batch3, Wc, bcr)
